# Initial kernel scaffold; baseline (speedup 1.0000x reference)
import jax
import jax.numpy as jnp
from jax.experimental import pallas as pl


def kernel(x, weights):
    raise NotImplementedError("write your pallas kernel here")


# speedup vs baseline: 15.6188x; 15.6188x over previous
import jax
import jax.numpy as jnp
from jax.experimental import pallas as pl

_ROWS, _COLS, _D = 64, 64, 32
_N = _ROWS * _COLS
_BB = 256


def _bmu_block(x_ref, wt_ref, out_ref):
    xb = x_ref[...]
    wt = wt_ref[...]
    wn2 = jnp.sum(wt * wt, axis=0, keepdims=True)
    dots = jnp.dot(
        xb, wt,
        preferred_element_type=jnp.float32,
        precision=jax.lax.Precision.HIGHEST,
    )
    s = wn2 - 2.0 * dots
    m = jnp.min(s, axis=1, keepdims=True)
    ii = jax.lax.broadcasted_iota(jnp.int32, s.shape, 1)
    idx = jnp.min(jnp.where(s <= m, ii, jnp.int32(_N)), axis=1, keepdims=True)
    out_ref[...] = jnp.concatenate([idx // _COLS, idx % _COLS], axis=1)


def kernel(x, weights):
    batch = x.shape[0]
    wt = weights.reshape(_N, _D).T
    return pl.pallas_call(
        _bmu_block,
        grid=(batch // _BB,),
        in_specs=[
            pl.BlockSpec((_BB, _D), lambda i: (i, 0)),
            pl.BlockSpec((_D, _N), lambda i: (0, 0)),
        ],
        out_specs=pl.BlockSpec((_BB, 2), lambda i: (i, 0)),
        out_shape=jax.ShapeDtypeStruct((batch, 2), jnp.int32),
    )(x, wt)


# speedup vs baseline: 15.7567x; 1.0088x over previous
import jax
import jax.numpy as jnp
from jax.experimental import pallas as pl

_ROWS, _COLS, _D = 64, 64, 32
_N = _ROWS * _COLS
_BB = 1024


def _bmu_block(x_ref, wt_ref, out_ref):
    xb = x_ref[...]
    wt = wt_ref[...]
    wn2 = jnp.sum(wt * wt, axis=0, keepdims=True)
    dots = jnp.dot(
        xb, wt,
        preferred_element_type=jnp.float32,
        precision=jax.lax.Precision.HIGHEST,
    )
    s = wn2 - 2.0 * dots
    m = jnp.min(s, axis=1, keepdims=True)
    ii = jax.lax.broadcasted_iota(jnp.int32, s.shape, 1)
    idx = jnp.min(jnp.where(s <= m, ii, jnp.int32(_N)), axis=1, keepdims=True)
    out_ref[...] = jnp.concatenate([idx // _COLS, idx % _COLS], axis=1)


def kernel(x, weights):
    batch = x.shape[0]
    wt = weights.reshape(_N, _D).T
    return pl.pallas_call(
        _bmu_block,
        grid=(batch // _BB,),
        in_specs=[
            pl.BlockSpec((_BB, _D), lambda i: (i, 0)),
            pl.BlockSpec((_D, _N), lambda i: (0, 0)),
        ],
        out_specs=pl.BlockSpec((_BB, 2), lambda i: (i, 0)),
        out_shape=jax.ShapeDtypeStruct((batch, 2), jnp.int32),
    )(x, wt)
